# baseline (speedup 1.0000x reference)
import functools

import jax
import jax.numpy as jnp
from jax import lax
from jax.experimental import pallas as pl
from jax.experimental.pallas import tpu as pltpu
from jax.experimental.pallas import tpu_sc as plsc

B = 16384
VOCAB = 4823
VPAD = 4864
VHALF = 2432
EMBED = 64
N_FIELDS = 26
HIDDEN = 256


def _prep_body(embt_ref, w4b_ref, pt_ref):
    ptf = lax.dot_general(
        w4b_ref[:], embt_ref[:], (((1,), (0,)), ((), ())),
        preferred_element_type=jnp.float32)
    lo = ptf[:, :VHALF]
    hi = ptf[:, VHALF:]
    lo_u = lax.bitcast_convert_type(
        lo.astype(jnp.bfloat16), jnp.uint16).astype(jnp.uint32)
    hi_u = lax.bitcast_convert_type(
        hi.astype(jnp.bfloat16), jnp.uint16).astype(jnp.uint32)
    pt_ref[:] = lax.bitcast_convert_type((hi_u << 16) | lo_u, jnp.int32)


def _mlp_body(xt_ref, w1_ref, w2_ref, w3_ref, w4a_ref, d_ref):
    w34 = lax.dot_general(
        w3_ref[:], w4a_ref[:], (((1,), (0,)), ((), ())),
        preferred_element_type=jnp.float32)
    h = jnp.maximum(
        lax.dot_general(w1_ref[:], xt_ref[:], (((0,), (0,)), ((), ())),
                        preferred_element_type=jnp.float32), 0.0)
    h = jnp.maximum(
        lax.dot_general(w2_ref[:], h, (((0,), (0,)), ((), ())),
                        preferred_element_type=jnp.float32), 0.0)
    d_ref[:] = jnp.sum(h * w34, axis=0)


def _combine_body(d_ref, s_ref, o_ref):
    o_ref[:] = jax.nn.sigmoid(d_ref[:] + s_ref[:])


def _make_sc_gather(num_workers, rows_per_w):
    mesh = plsc.VectorSubcoreMesh(core_axis_name="c", subcore_axis_name="s")
    groups = rows_per_w // 16

    @functools.partial(
        pl.kernel,
        mesh=mesh,
        out_type=jax.ShapeDtypeStruct((B,), jnp.float32),
        compiler_params=pltpu.CompilerParams(
            needs_layout_passes=False, use_tc_tiling_on_sc=False,
            disable_bounds_checks=True, skip_device_barrier=True),
        scratch_types=[
            pltpu.VMEM((N_FIELDS, VHALF), jnp.int32),
            pltpu.VMEM((N_FIELDS, rows_per_w), jnp.int32),
            pltpu.VMEM((rows_per_w,), jnp.float32),
            pltpu.SemaphoreType.DMA,
            pltpu.SemaphoreType.DMA,
        ],
    )
    def sc_gather(tab_hbm, idx_hbm, out_hbm, tab_v, idx_v, out_v,
                  tab_sem, idx_sem):
        nc = 2
        wid = lax.axis_index("s") * nc + lax.axis_index("c")
        base = wid * rows_per_w
        idx_cps = [
            pltpu.async_copy(idx_hbm.at[f, pl.ds(base, rows_per_w)],
                             idx_v.at[f], idx_sem)
            for f in range(N_FIELDS)
        ]
        tab_cps = [
            pltpu.async_copy(tab_hbm.at[f], tab_v.at[f], tab_sem)
            for f in range(N_FIELDS)
        ]

        for f in range(N_FIELDS):
            idx_cps[f].wait()
            tab_cps[f].wait()
            fv = jnp.full((16,), f, jnp.int32)

            def group_body(g, _, f=f, fv=fv):
                iv = idx_v[f, pl.ds(g * 16, 16)]
                hi_sel = iv >= VHALF
                u = jnp.where(hi_sel, iv - VHALF, iv)
                w = plsc.load_gather(tab_v, [fv, u])
                bits = jnp.where(hi_sel, w & jnp.int32(-65536), w << 16)
                val = plsc.bitcast(bits, jnp.float32)
                sl = pl.ds(g * 16, 16)
                if f == 0:
                    out_v[sl] = val
                else:
                    out_v[sl] = out_v[sl] + val
                return 0

            lax.fori_loop(0, groups, group_body, 0)
        pltpu.sync_copy(out_v, out_hbm.at[pl.ds(base, rows_per_w)])

    return sc_gather


def kernel(dense_input, sparse_input, emb_table, W1, W2, W3, W4):
    w4a = W4[:HIDDEN]
    w4b = W4[HIDDEN:, 0].reshape(N_FIELDS, EMBED)
    embt = emb_table.T

    pt = pl.pallas_call(
        _prep_body,
        grid=(1,),
        in_specs=[
            pl.BlockSpec((64, VPAD), lambda i: (0, 0)),
            pl.BlockSpec((N_FIELDS, EMBED), lambda i: (0, 0)),
        ],
        out_specs=pl.BlockSpec((N_FIELDS, VHALF), lambda i: (0, 0)),
        out_shape=jax.ShapeDtypeStruct((N_FIELDS, VHALF), jnp.int32),
        compiler_params=pltpu.CompilerParams(skip_device_barrier=True),
    )(embt, w4b)

    info = plsc.get_sparse_core_info()
    num_workers = info.num_cores * info.num_subcores
    rows_per_w = B // num_workers

    sc_gather = _make_sc_gather(num_workers, rows_per_w)
    s = sc_gather(pt, sparse_input.T)

    bm = 2048
    d = pl.pallas_call(
        _mlp_body,
        grid=(B // bm,),
        in_specs=[
            pl.BlockSpec((13, bm), lambda i: (0, i)),
            pl.BlockSpec((13, HIDDEN), lambda i: (0, 0)),
            pl.BlockSpec((HIDDEN, HIDDEN), lambda i: (0, 0)),
            pl.BlockSpec((HIDDEN, HIDDEN), lambda i: (0, 0)),
            pl.BlockSpec((HIDDEN, 1), lambda i: (0, 0)),
        ],
        out_specs=pl.BlockSpec((bm,), lambda i: (i,)),
        out_shape=jax.ShapeDtypeStruct((B,), jnp.float32),
        compiler_params=pltpu.CompilerParams(
            fuse_transposed_lhs_in_matmul=True, skip_device_barrier=True),
    )(dense_input.T, W1, W2, W3, w4a)

    out = pl.pallas_call(
        _combine_body,
        grid=(1,),
        in_specs=[
            pl.BlockSpec((B,), lambda i: (0,)),
            pl.BlockSpec((B,), lambda i: (0,)),
        ],
        out_specs=pl.BlockSpec((B,), lambda i: (0,)),
        out_shape=jax.ShapeDtypeStruct((B,), jnp.float32),
        compiler_params=pltpu.CompilerParams(skip_device_barrier=True),
    )(d, s)
    return out.reshape(B, 1)


# speedup vs baseline: 1.0002x; 1.0002x over previous
import functools

import jax
import jax.numpy as jnp
from jax import lax
from jax.experimental import pallas as pl
from jax.experimental.pallas import tpu as pltpu
from jax.experimental.pallas import tpu_sc as plsc

B = 16384
VOCAB = 4823
VPAD = 4864
VHALF = 2432
EMBED = 64
N_FIELDS = 26
HIDDEN = 256


def _prep_body(embt_ref, w4b_ref, pt_ref):
    ptf = lax.dot_general(
        w4b_ref[:], embt_ref[:], (((1,), (0,)), ((), ())),
        preferred_element_type=jnp.float32)
    lo = ptf[:, :VHALF]
    hi = ptf[:, VHALF:]
    lo_u = lax.bitcast_convert_type(
        lo.astype(jnp.bfloat16), jnp.uint16).astype(jnp.uint32)
    hi_u = lax.bitcast_convert_type(
        hi.astype(jnp.bfloat16), jnp.uint16).astype(jnp.uint32)
    pt_ref[:] = lax.bitcast_convert_type((hi_u << 16) | lo_u, jnp.int32)


def _mlp_body(xt_ref, w1_ref, w2_ref, w3_ref, w4a_ref, d_ref):
    w34 = lax.dot_general(
        w3_ref[:], w4a_ref[:], (((1,), (0,)), ((), ())),
        preferred_element_type=jnp.float32)
    h = jnp.maximum(
        lax.dot_general(w1_ref[:], xt_ref[:], (((0,), (0,)), ((), ())),
                        preferred_element_type=jnp.float32), 0.0)
    h = jnp.maximum(
        lax.dot_general(w2_ref[:], h, (((0,), (0,)), ((), ())),
                        preferred_element_type=jnp.float32), 0.0)
    d_ref[:] = jnp.sum(h * w34, axis=0)


def _combine_body(d_ref, s_ref, o_ref):
    o_ref[:] = jax.nn.sigmoid(d_ref[:] + s_ref[:])


def _make_sc_gather(num_workers, rows_per_w):
    mesh = plsc.VectorSubcoreMesh(core_axis_name="c", subcore_axis_name="s")
    groups = rows_per_w // 16

    @functools.partial(
        pl.kernel,
        mesh=mesh,
        out_type=jax.ShapeDtypeStruct((B,), jnp.float32),
        compiler_params=pltpu.CompilerParams(
            needs_layout_passes=False, use_tc_tiling_on_sc=False,
            disable_bounds_checks=True, skip_device_barrier=True),
        scratch_types=[
            pltpu.VMEM((N_FIELDS, VHALF), jnp.int32),
            pltpu.VMEM((N_FIELDS, rows_per_w), jnp.int32),
            pltpu.VMEM((rows_per_w,), jnp.float32),
            pltpu.SemaphoreType.DMA,
            pltpu.SemaphoreType.DMA,
        ],
    )
    def sc_gather(tab_hbm, idx_hbm, out_hbm, tab_v, idx_v, out_v,
                  tab_sem, idx_sem):
        nc = 2
        wid = lax.axis_index("s") * nc + lax.axis_index("c")
        base = wid * rows_per_w
        idx_cps = [
            pltpu.async_copy(idx_hbm.at[f, pl.ds(base, rows_per_w)],
                             idx_v.at[f], idx_sem)
            for f in range(N_FIELDS)
        ]
        tab_cps = [
            pltpu.async_copy(tab_hbm.at[f], tab_v.at[f], tab_sem)
            for f in range(N_FIELDS)
        ]

        for f in range(N_FIELDS):
            idx_cps[f].wait()
            tab_cps[f].wait()
            fv = jnp.full((16,), f, jnp.int32)

            def group_body(g, _, f=f, fv=fv):
                iv = idx_v[f, pl.ds(g * 16, 16)]
                hi_sel = iv >= VHALF
                u = jnp.where(hi_sel, iv - VHALF, iv)
                w = plsc.load_gather(tab_v, [fv, u])
                bits = jnp.where(hi_sel, w & jnp.int32(-65536), w << 16)
                val = plsc.bitcast(bits, jnp.float32)
                sl = pl.ds(g * 16, 16)
                if f == 0:
                    out_v[sl] = val
                else:
                    out_v[sl] = out_v[sl] + val
                return 0

            lax.fori_loop(0, groups, group_body, 0)
        pltpu.sync_copy(out_v, out_hbm.at[pl.ds(base, rows_per_w)])

    return sc_gather


def kernel(dense_input, sparse_input, emb_table, W1, W2, W3, W4):
    w4a = W4[:HIDDEN]
    w4b = W4[HIDDEN:, 0].reshape(N_FIELDS, EMBED)
    embt = emb_table.T

    pt = pl.pallas_call(
        _prep_body,
        grid=(1,),
        in_specs=[
            pl.BlockSpec((64, VPAD), lambda i: (0, 0)),
            pl.BlockSpec((N_FIELDS, EMBED), lambda i: (0, 0)),
        ],
        out_specs=pl.BlockSpec((N_FIELDS, VHALF), lambda i: (0, 0)),
        out_shape=jax.ShapeDtypeStruct((N_FIELDS, VHALF), jnp.int32),
        compiler_params=pltpu.CompilerParams(skip_device_barrier=True),
    )(embt, w4b)

    info = plsc.get_sparse_core_info()
    num_workers = info.num_cores * info.num_subcores
    rows_per_w = B // num_workers

    sc_gather = _make_sc_gather(num_workers, rows_per_w)
    s = sc_gather(pt, sparse_input.T)

    bm = 4096
    d = pl.pallas_call(
        _mlp_body,
        grid=(B // bm,),
        in_specs=[
            pl.BlockSpec((13, bm), lambda i: (0, i)),
            pl.BlockSpec((13, HIDDEN), lambda i: (0, 0)),
            pl.BlockSpec((HIDDEN, HIDDEN), lambda i: (0, 0)),
            pl.BlockSpec((HIDDEN, HIDDEN), lambda i: (0, 0)),
            pl.BlockSpec((HIDDEN, 1), lambda i: (0, 0)),
        ],
        out_specs=pl.BlockSpec((bm,), lambda i: (i,)),
        out_shape=jax.ShapeDtypeStruct((B,), jnp.float32),
        compiler_params=pltpu.CompilerParams(
            fuse_transposed_lhs_in_matmul=True, skip_device_barrier=True),
    )(dense_input.T, W1, W2, W3, w4a)

    out = pl.pallas_call(
        _combine_body,
        grid=(1,),
        in_specs=[
            pl.BlockSpec((B,), lambda i: (0,)),
            pl.BlockSpec((B,), lambda i: (0,)),
        ],
        out_specs=pl.BlockSpec((B,), lambda i: (0,)),
        out_shape=jax.ShapeDtypeStruct((B,), jnp.float32),
        compiler_params=pltpu.CompilerParams(skip_device_barrier=True),
    )(d, s)
    return out.reshape(B, 1)


# speedup vs baseline: 1.0084x; 1.0083x over previous
import functools

import jax
import jax.numpy as jnp
from jax import lax
from jax.experimental import pallas as pl
from jax.experimental.pallas import tpu as pltpu
from jax.experimental.pallas import tpu_sc as plsc

B = 16384
VOCAB = 4823
VPAD = 4864
VHALF = 2432
EMBED = 64
N_FIELDS = 26
HIDDEN = 256


def _prep_body(embt_ref, w4b_ref, pt_ref):
    ptf = lax.dot_general(
        w4b_ref[:], embt_ref[:], (((1,), (0,)), ((), ())),
        preferred_element_type=jnp.float32)
    lo = ptf[:, :VHALF]
    hi = ptf[:, VHALF:]
    lo_u = lax.bitcast_convert_type(
        lo.astype(jnp.bfloat16), jnp.uint16).astype(jnp.uint32)
    hi_u = lax.bitcast_convert_type(
        hi.astype(jnp.bfloat16), jnp.uint16).astype(jnp.uint32)
    pt_ref[:] = lax.bitcast_convert_type((hi_u << 16) | lo_u, jnp.int32)


def _mlp_body(xt_ref, w1_ref, w2_ref, w3_ref, w4a_ref, d_ref):
    w34 = lax.dot_general(
        w3_ref[:], w4a_ref[:], (((1,), (0,)), ((), ())),
        preferred_element_type=jnp.float32)
    h = jnp.maximum(
        lax.dot_general(w1_ref[:], xt_ref[:], (((0,), (0,)), ((), ())),
                        preferred_element_type=jnp.float32), 0.0)
    h = jnp.maximum(
        lax.dot_general(w2_ref[:], h, (((0,), (0,)), ((), ())),
                        preferred_element_type=jnp.float32), 0.0)
    d_ref[:] = jnp.sum(h * w34, axis=0)


def _combine_body(d_ref, s_ref, o_ref):
    o_ref[:] = jax.nn.sigmoid(d_ref[:] + s_ref[:])


def _make_sc_gather(num_workers, rows_per_w):
    mesh = plsc.VectorSubcoreMesh(core_axis_name="c", subcore_axis_name="s")
    groups = rows_per_w // 16

    @functools.partial(
        pl.kernel,
        mesh=mesh,
        out_type=jax.ShapeDtypeStruct((B,), jnp.float32),
        compiler_params=pltpu.CompilerParams(
            needs_layout_passes=False, use_tc_tiling_on_sc=False,
            disable_bounds_checks=True, skip_device_barrier=True),
        scratch_types=[
            pltpu.VMEM((N_FIELDS, VHALF), jnp.int32),
            pltpu.VMEM((N_FIELDS, rows_per_w), jnp.int32),
            pltpu.VMEM((rows_per_w,), jnp.float32),
            pltpu.SemaphoreType.DMA,
            pltpu.SemaphoreType.DMA,
        ],
    )
    def sc_gather(tab_hbm, idx_hbm, out_hbm, tab_v, idx_v, out_v,
                  tab_sem, idx_sem):
        nc = 2
        wid = lax.axis_index("s") * nc + lax.axis_index("c")
        base = wid * rows_per_w
        idx_cps = [
            pltpu.async_copy(idx_hbm.at[f, pl.ds(base, rows_per_w)],
                             idx_v.at[f], idx_sem)
            for f in range(N_FIELDS)
        ]
        tab_cps = [
            pltpu.async_copy(tab_hbm.at[f], tab_v.at[f], tab_sem)
            for f in range(N_FIELDS)
        ]

        for f in range(N_FIELDS):
            idx_cps[f].wait()
            tab_cps[f].wait()
            fv = jnp.full((16,), f, jnp.int32)

            def group_body(g, _, f=f, fv=fv):
                iv = idx_v[f, pl.ds(g * 16, 16)]
                hi_sel = iv >= VHALF
                u = jnp.where(hi_sel, iv - VHALF, iv)
                w = plsc.load_gather(tab_v, [fv, u])
                bits = jnp.where(hi_sel, w & jnp.int32(-65536), w << 16)
                val = plsc.bitcast(bits, jnp.float32)
                sl = pl.ds(g * 16, 16)
                if f == 0:
                    out_v[sl] = val
                else:
                    out_v[sl] = out_v[sl] + val
                return 0

            lax.fori_loop(0, groups, group_body, 0)
        pltpu.sync_copy(out_v, out_hbm.at[pl.ds(base, rows_per_w)])

    return sc_gather


def kernel(dense_input, sparse_input, emb_table, W1, W2, W3, W4):
    w4a = W4[:HIDDEN]
    w4b = W4[HIDDEN:, 0].reshape(N_FIELDS, EMBED)
    embt = emb_table.T

    pt = pl.pallas_call(
        _prep_body,
        grid=(1,),
        in_specs=[
            pl.BlockSpec((64, VPAD), lambda i: (0, 0)),
            pl.BlockSpec((N_FIELDS, EMBED), lambda i: (0, 0)),
        ],
        out_specs=pl.BlockSpec((N_FIELDS, VHALF), lambda i: (0, 0)),
        out_shape=jax.ShapeDtypeStruct((N_FIELDS, VHALF), jnp.int32),
        compiler_params=pltpu.CompilerParams(skip_device_barrier=True),
    )(embt, w4b)

    info = plsc.get_sparse_core_info()
    num_workers = info.num_cores * info.num_subcores
    rows_per_w = B // num_workers

    sc_gather = _make_sc_gather(num_workers, rows_per_w)
    s = sc_gather(pt, sparse_input.T)

    bm = 8192
    d = pl.pallas_call(
        _mlp_body,
        grid=(B // bm,),
        in_specs=[
            pl.BlockSpec((13, bm), lambda i: (0, i)),
            pl.BlockSpec((13, HIDDEN), lambda i: (0, 0)),
            pl.BlockSpec((HIDDEN, HIDDEN), lambda i: (0, 0)),
            pl.BlockSpec((HIDDEN, HIDDEN), lambda i: (0, 0)),
            pl.BlockSpec((HIDDEN, 1), lambda i: (0, 0)),
        ],
        out_specs=pl.BlockSpec((bm,), lambda i: (i,)),
        out_shape=jax.ShapeDtypeStruct((B,), jnp.float32),
        compiler_params=pltpu.CompilerParams(
            fuse_transposed_lhs_in_matmul=True, skip_device_barrier=True),
    )(dense_input.T, W1, W2, W3, w4a)

    out = pl.pallas_call(
        _combine_body,
        grid=(1,),
        in_specs=[
            pl.BlockSpec((B,), lambda i: (0,)),
            pl.BlockSpec((B,), lambda i: (0,)),
        ],
        out_specs=pl.BlockSpec((B,), lambda i: (0,)),
        out_shape=jax.ShapeDtypeStruct((B,), jnp.float32),
        compiler_params=pltpu.CompilerParams(skip_device_barrier=True),
    )(d, s)
    return out.reshape(B, 1)
